# baseline (speedup 1.0000x reference)
import functools

import jax
import jax.numpy as jnp
from jax import lax
from jax.experimental import pallas as pl
from jax.experimental.pallas import tpu as pltpu
from jax.experimental.pallas import tpu_sc as plsc

_VOCAB = 100000
_D = 128
_MD = 2048
_L = 16
_NC = 2
_NS = 16
_NW = _NC * _NS
_N = 4 * 4096
_K = 4
_CHUNK = _N // _K
_TPW = _CHUNK // _NW
_TOK_BLK = 512
_NBLK = _CHUNK // _TOK_BLK


def _mod_vocab(v):
  q = (v.astype(jnp.float32) * (1.0 / _VOCAB)).astype(jnp.int32)
  r = v - q * _VOCAB
  r = jnp.where(r < 0, r + _VOCAB, r)
  return jnp.where(r >= _VOCAB, r - _VOCAB, r)


def _make_sc_gather():
  mesh = plsc.VectorSubcoreMesh(
      core_axis_name="c", subcore_axis_name="s",
      num_cores=_NC, num_subcores=_NS)

  @functools.partial(
      pl.kernel,
      out_type=[jax.ShapeDtypeStruct((_CHUNK, _D), jnp.float32),
                jax.ShapeDtypeStruct((_CHUNK, _D), jnp.float32)],
      mesh=mesh,
      scratch_types=[
          pltpu.VMEM((_TPW + 8,), jnp.int32),
          pltpu.VMEM((_TPW,), jnp.int32),
          pltpu.VMEM((_TPW,), jnp.int32),
          pltpu.VMEM((_TPW, _D), jnp.float32),
          pltpu.VMEM((_TPW, _D), jnp.float32),
          pltpu.SemaphoreType.DMA,
          pltpu.SemaphoreType.DMA,
          pltpu.SemaphoreType.DMA,
      ],
  )
  def k(xp_hbm, e1_hbm, e2_hbm, o1_hbm, o2_hbm,
        xe_v, i1_v, i2_v, r1_v, r2_v, s1, s2, s3):
    wid = lax.axis_index("s") * _NC + lax.axis_index("c")
    base = pl.multiple_of(wid * _TPW, 8)
    pltpu.sync_copy(xp_hbm.at[pl.ds(base, _TPW + 8)], xe_v)

    lanes = lax.iota(jnp.int32, _L)
    head = xe_v[pl.ds(0, _L)]
    head = jnp.where(lanes + wid * _L == 7, jnp.zeros((_L,), jnp.int32),
                     head)
    xe_v[pl.ds(0, _L)] = head

    def hash_step(i, _):
      t0 = i * _L
      prev = xe_v[pl.ds(t0 + 7, _L)]
      cur = xe_v[pl.ds(t0 + 8, _L)]
      i1_v[pl.ds(t0, _L)] = _mod_vocab(prev * 1024 + cur)
      i2_v[pl.ds(t0, _L)] = _mod_vocab(prev + cur * 31)
      return _
    lax.fori_loop(0, _TPW // _L, hash_step, None)
    g1 = pltpu.async_copy(e1_hbm.at[i1_v], r1_v, s1)
    g2 = pltpu.async_copy(e2_hbm.at[i2_v], r2_v, s2)
    g1.wait()
    w1 = pltpu.async_copy(r1_v, o1_hbm.at[pl.ds(base, _TPW)], s3)
    g2.wait()
    pltpu.sync_copy(r2_v, o2_hbm.at[pl.ds(base, _TPW)])
    w1.wait()

  return k


def _tc_matmul_chunk(e1_c, e2_c, w_b, out_buf, c):
  def body(e1_ref, e2_ref, w_ref, *rest):
    o_ref = rest[-1]
    a = jnp.concatenate([e1_ref[...], e2_ref[...]], axis=-1)
    o_ref[...] = lax.dot_general(
        a.astype(jnp.bfloat16), w_ref[...], (((1,), (1,)), ((), ())),
        preferred_element_type=jnp.float32)

  in_specs = [
      pl.BlockSpec((_TOK_BLK, _D), lambda i: (i, 0)),
      pl.BlockSpec((_TOK_BLK, _D), lambda i: (i, 0)),
      pl.BlockSpec((_MD, 2 * _D), lambda i: (0, 0)),
  ]
  operands = [e1_c, e2_c, w_b]
  aliases = {}
  if out_buf is not None:
    in_specs.append(pl.BlockSpec((8, 128), lambda i: (0, 0)))
    operands.append(out_buf)
    aliases = {3: 0}
  return pl.pallas_call(
      body,
      grid=(_NBLK,),
      in_specs=in_specs,
      out_specs=pl.BlockSpec((_TOK_BLK, _MD),
                             lambda i, c=c: (i + c * _NBLK, 0)),
      out_shape=jax.ShapeDtypeStruct((_N, _MD), jnp.float32),
      input_output_aliases=aliases,
  )(*operands)


def kernel(x, embed1, embed2, proj_w, gate):
  del gate
  b, s = x.shape
  xp = jnp.concatenate([jnp.zeros((8,), jnp.int32), x.reshape(-1)])
  w_b = proj_w.astype(jnp.bfloat16)
  sc = _make_sc_gather()

  es = [sc(lax.dynamic_slice(xp, (c * _CHUNK,), (_CHUNK + 8,)),
           embed1, embed2)
        for c in range(_K)]
  out = None
  for c, (e1_c, e2_c) in enumerate(es):
    out = _tc_matmul_chunk(e1_c, e2_c, w_b, out, c)
  return out.reshape(b, s, _MD)


# speedup vs baseline: 1.0658x; 1.0658x over previous
import functools

import jax
import jax.numpy as jnp
from jax import lax
from jax.experimental import pallas as pl
from jax.experimental.pallas import tpu as pltpu
from jax.experimental.pallas import tpu_sc as plsc

_VOCAB = 100000
_D = 128
_MD = 2048
_L = 16
_NC = 2
_NS = 16
_NW = _NC * _NS
_N = 4 * 4096
_K = 4
_CHUNK = _N // _K
_TPW = _CHUNK // _NW
_TOK_BLK = 1024
_NBLK = _CHUNK // _TOK_BLK


def _mod_vocab(v):
  q = (v.astype(jnp.float32) * (1.0 / _VOCAB)).astype(jnp.int32)
  r = v - q * _VOCAB
  r = jnp.where(r < 0, r + _VOCAB, r)
  return jnp.where(r >= _VOCAB, r - _VOCAB, r)


def _make_sc_gather(c):
  mesh = plsc.VectorSubcoreMesh(
      core_axis_name="c", subcore_axis_name="s",
      num_cores=_NC, num_subcores=_NS)

  @functools.partial(
      pl.kernel,
      out_type=[jax.ShapeDtypeStruct((_CHUNK, _D), jnp.float32),
                jax.ShapeDtypeStruct((_CHUNK, _D), jnp.float32)],
      mesh=mesh,
      scratch_types=[
          pltpu.VMEM((_TPW + 8,), jnp.int32),
          pltpu.VMEM((_TPW,), jnp.int32),
          pltpu.VMEM((_TPW,), jnp.int32),
          pltpu.VMEM((_TPW, _D), jnp.float32),
          pltpu.VMEM((_TPW, _D), jnp.float32),
          pltpu.SemaphoreType.DMA,
          pltpu.SemaphoreType.DMA,
          pltpu.SemaphoreType.DMA,
      ],
  )
  def k(xp_hbm, e1_hbm, e2_hbm, o1_hbm, o2_hbm,
        xe_v, i1_v, i2_v, r1_v, r2_v, s1, s2, s3):
    wid = lax.axis_index("s") * _NC + lax.axis_index("c")
    base = pl.multiple_of(wid * _TPW, 8)
    pltpu.sync_copy(xp_hbm.at[pl.ds(c * _CHUNK + base, _TPW + 8)], xe_v)

    lanes = lax.iota(jnp.int32, _L)
    head = xe_v[pl.ds(0, _L)]
    head = jnp.where(lanes + wid * _L == 7, jnp.zeros((_L,), jnp.int32),
                     head)
    xe_v[pl.ds(0, _L)] = head

    def hash_step(i, _):
      t0 = i * _L
      prev = xe_v[pl.ds(t0 + 7, _L)]
      cur = xe_v[pl.ds(t0 + 8, _L)]
      i1_v[pl.ds(t0, _L)] = _mod_vocab(prev * 1024 + cur)
      i2_v[pl.ds(t0, _L)] = _mod_vocab(prev + cur * 31)
      return _
    lax.fori_loop(0, _TPW // _L, hash_step, None)
    g1 = pltpu.async_copy(e1_hbm.at[i1_v], r1_v, s1)
    g2 = pltpu.async_copy(e2_hbm.at[i2_v], r2_v, s2)
    g1.wait()
    w1 = pltpu.async_copy(r1_v, o1_hbm.at[pl.ds(base, _TPW)], s3)
    g2.wait()
    pltpu.sync_copy(r2_v, o2_hbm.at[pl.ds(base, _TPW)])
    w1.wait()

  return k


def _tc_matmul_chunk(e1_c, e2_c, w_b, out_buf, c):
  def body(e1_ref, e2_ref, w_ref, *rest):
    o_ref = rest[-1]
    a = jnp.concatenate([e1_ref[...], e2_ref[...]], axis=-1)
    o_ref[...] = lax.dot_general(
        a.astype(jnp.bfloat16), w_ref[...], (((1,), (1,)), ((), ())),
        preferred_element_type=jnp.float32)

  in_specs = [
      pl.BlockSpec((_TOK_BLK, _D), lambda i: (i, 0)),
      pl.BlockSpec((_TOK_BLK, _D), lambda i: (i, 0)),
      pl.BlockSpec((_MD, 2 * _D), lambda i: (0, 0)),
  ]
  operands = [e1_c, e2_c, w_b]
  aliases = {}
  if out_buf is not None:
    in_specs.append(pl.BlockSpec((8, 128), lambda i: (0, 0)))
    operands.append(out_buf)
    aliases = {3: 0}
  return pl.pallas_call(
      body,
      grid=(_NBLK,),
      in_specs=in_specs,
      out_specs=pl.BlockSpec((_TOK_BLK, _MD),
                             lambda i, c=c: (i + c * _NBLK, 0)),
      out_shape=jax.ShapeDtypeStruct((_N, _MD), jnp.float32),
      input_output_aliases=aliases,
  )(*operands)


def kernel(x, embed1, embed2, proj_w, gate):
  del gate
  b, s = x.shape
  xp = jnp.concatenate([jnp.zeros((8,), jnp.int32), x.reshape(-1)])
  w_b = proj_w.astype(jnp.bfloat16)

  es = [_make_sc_gather(c)(xp, embed1, embed2) for c in range(_K)]
  out = None
  for c, (e1_c, e2_c) in enumerate(es):
    out = _tc_matmul_chunk(e1_c, e2_c, w_b, out, c)
  return out.reshape(b, s, _MD)
